# Initial kernel scaffold; baseline (speedup 1.0000x reference)
import jax
import jax.numpy as jnp
from jax.experimental import pallas as pl


def kernel(nodes, neigh_idx, features, weight):
    raise NotImplementedError("write your pallas kernel here")


# speedup vs baseline: 1.1062x; 1.1062x over previous
import functools

import jax
import jax.numpy as jnp
from jax import lax
from jax.experimental import pallas as pl
from jax.experimental.pallas import tpu as pltpu
from jax.experimental.pallas import tpu_sc as plsc

N_NODES = 50000
D = 512
E = 512
B = 10000
S = 10

NC, NS, L = 2, 16, 16
NW = NC * NS
B_PAD = 10240
RPW = B_PAD // NW
K = 16
NCHUNK = RPW // K
H = (K * S) // 2


def _sc_gather_body(nodes_hbm, neigh_hbm, feat_hbm,
                    self_out, mean_out,
                    idx_s, idx_n0, idx_n1,
                    self_buf, nbuf0, nbuf1, acc_buf,
                    sem_s, sem_n0, sem_n1):
    wid = lax.axis_index("s") * NC + lax.axis_index("c")
    base = wid * RPW

    def chunk_body(ch, _):
        row0 = base + ch * K
        pltpu.sync_copy(nodes_hbm.at[pl.ds(row0, K)], idx_s)
        pltpu.sync_copy(neigh_hbm.at[pl.ds(row0 * S, H)], idx_n0)
        pltpu.sync_copy(neigh_hbm.at[pl.ds(row0 * S + H, H)], idx_n1)
        cp_s = pltpu.async_copy(feat_hbm.at[idx_s], self_buf, sem_s)
        cp_n0 = pltpu.async_copy(feat_hbm.at[idx_n0], nbuf0, sem_n0)
        cp_n1 = pltpu.async_copy(feat_hbm.at[idx_n1], nbuf1, sem_n1)
        cp_s.wait()
        pltpu.sync_copy(self_buf, self_out.at[pl.ds(row0, K)])
        cp_n0.wait()
        cp_n1.wait()

        inv_s = jnp.float32(1.0 / S)

        def rows_from(buf, r_off):
            def row_body(r, _):
                rbase = r * S
                for c in range(D // L):
                    sl = pl.ds(c * L, L)
                    a = buf[rbase, sl]
                    for j in range(1, S):
                        a = a + buf[rbase + j, sl]
                    acc_buf[r + r_off, sl] = a * inv_s
                return 0
            lax.fori_loop(0, K // 2, row_body, 0)

        rows_from(nbuf0, 0)
        rows_from(nbuf1, K // 2)
        pltpu.sync_copy(acc_buf, mean_out.at[pl.ds(row0, K)])
        return 0

    lax.fori_loop(0, NCHUNK, chunk_body, 0)


@jax.jit
def _sc_gather(nodes_pad, neigh_pad_flat, features):
    mesh = plsc.VectorSubcoreMesh(core_axis_name="c", subcore_axis_name="s")
    f = pl.kernel(
        _sc_gather_body,
        out_type=(
            jax.ShapeDtypeStruct((B_PAD, D), jnp.float32),
            jax.ShapeDtypeStruct((B_PAD, D), jnp.float32),
        ),
        mesh=mesh,
        scratch_types=[
            pltpu.VMEM((K,), jnp.int32),
            pltpu.VMEM((H,), jnp.int32),
            pltpu.VMEM((H,), jnp.int32),
            pltpu.VMEM((K, D), jnp.float32),
            pltpu.VMEM((H, D), jnp.float32),
            pltpu.VMEM((H, D), jnp.float32),
            pltpu.VMEM((K, D), jnp.float32),
            pltpu.SemaphoreType.DMA,
            pltpu.SemaphoreType.DMA,
            pltpu.SemaphoreType.DMA,
        ],
    )
    return f(nodes_pad, neigh_pad_flat, features)


def _mm_body(ws_ref, wn_ref, self_ref, mean_ref, out_ref):
    a = lax.dot_general(ws_ref[...], self_ref[...],
                        (((1,), (1,)), ((), ())),
                        preferred_element_type=jnp.float32)
    b = lax.dot_general(wn_ref[...], mean_ref[...],
                        (((1,), (1,)), ((), ())),
                        preferred_element_type=jnp.float32)
    out_ref[...] = jnp.maximum(a + b, 0.0)


BT = 512


@jax.jit
def _tc_matmul(ws, wn, self_f, mean_f):
    grid = (B_PAD // BT,)
    return pl.pallas_call(
        _mm_body,
        grid=grid,
        in_specs=[
            pl.BlockSpec((E, D), lambda i: (0, 0)),
            pl.BlockSpec((E, D), lambda i: (0, 0)),
            pl.BlockSpec((BT, D), lambda i: (i, 0)),
            pl.BlockSpec((BT, D), lambda i: (i, 0)),
        ],
        out_specs=pl.BlockSpec((E, BT), lambda i: (0, i)),
        out_shape=jax.ShapeDtypeStruct((E, B_PAD), jnp.float32),
        compiler_params=pltpu.CompilerParams(
            dimension_semantics=("arbitrary",)),
    )(ws, wn, self_f, mean_f)


def kernel(nodes, neigh_idx, features, weight):
    nodes = nodes.astype(jnp.int32)
    neigh_idx = neigh_idx.astype(jnp.int32)
    nodes_pad = jnp.pad(nodes, (0, B_PAD - B))
    neigh_pad = jnp.pad(neigh_idx, ((0, B_PAD - B), (0, 0))).reshape(-1)
    self_f, mean_f = _sc_gather(nodes_pad, neigh_pad, features)
    ws = weight[:, :D]
    wn = weight[:, D:]
    out = _tc_matmul(ws, wn, self_f, mean_f)
    return out[:, :B]


# speedup vs baseline: 1.5765x; 1.4252x over previous
import jax
import jax.numpy as jnp
from jax import lax
from jax.experimental import pallas as pl
from jax.experimental.pallas import tpu as pltpu
from jax.experimental.pallas import tpu_sc as plsc

N_NODES = 50000
D = 512
E = 512
B = 10000
S = 10

NC, NS, L = 2, 16, 16
NW = NC * NS
B_PAD = 10240
RPW = B_PAD // NW
K = 8
NCHUNK = RPW // K
KS = K * S


def _sc_gather_body(nodes_hbm, neigh_hbm, feat_hbm,
                    self_out, mean_out,
                    sidx, nidx,
                    self_buf0, self_buf1, nbuf0, nbuf1, acc0, acc1,
                    sem_gs0, sem_gs1, sem_gn0, sem_gn1,
                    sem_w0, sem_w1, sem_ws0, sem_ws1):
    wid = lax.axis_index("s") * NC + lax.axis_index("c")
    base = wid * RPW
    self_bufs = (self_buf0, self_buf1)
    nbufs = (nbuf0, nbuf1)
    accs = (acc0, acc1)
    sem_gs = (sem_gs0, sem_gs1)
    sem_gn = (sem_gn0, sem_gn1)
    sem_w = (sem_w0, sem_w1)
    sem_ws = (sem_ws0, sem_ws1)

    pltpu.sync_copy(nodes_hbm.at[wid], sidx)
    pltpu.sync_copy(neigh_hbm.at[wid], nidx)

    inv_s = jnp.float32(1.0 / S)

    def issue_gathers(ch, par):
        pltpu.async_copy(feat_hbm.at[sidx.at[ch]], self_bufs[par], sem_gs[par])
        pltpu.async_copy(feat_hbm.at[nidx.at[ch]], nbufs[par], sem_gn[par])

    issue_gathers(0, 0)
    issue_gathers(1, 1)

    def pair_body(p, _):
        for par in range(2):
            ch = p * 2 + par
            row0 = base + ch * K
            sbuf, nbuf, acc = self_bufs[par], nbufs[par], accs[par]
            pltpu.make_async_copy(feat_hbm.at[sidx.at[ch]], sbuf,
                                  sem_gs[par]).wait()
            pltpu.make_async_copy(feat_hbm.at[nidx.at[ch]], nbuf,
                                  sem_gn[par]).wait()
            pltpu.async_copy(sbuf, self_out.at[pl.ds(row0, K)], sem_ws[par])
            @pl.when(ch >= 2)
            def _():
                pltpu.make_async_copy(
                    acc, mean_out.at[pl.ds(row0, K)], sem_w[par]).wait()

            def row_body(r, _):
                rbase = r * S
                for c in range(D // L):
                    sl = pl.ds(c * L, L)
                    a = nbuf[rbase, sl]
                    for j in range(1, S):
                        a = a + nbuf[rbase + j, sl]
                    acc[r, sl] = a * inv_s
                return 0
            lax.fori_loop(0, K, row_body, 0)

            pltpu.async_copy(acc, mean_out.at[pl.ds(row0, K)], sem_w[par])

            @pl.when(ch + 2 < NCHUNK)
            def _():
                pltpu.make_async_copy(sbuf, self_out.at[pl.ds(row0, K)],
                                      sem_ws[par]).wait()
                issue_gathers(ch + 2, par)
        return 0

    lax.fori_loop(0, NCHUNK // 2, pair_body, 0)

    for par in range(2):
        pltpu.make_async_copy(accs[par], mean_out.at[pl.ds(0, K)],
                              sem_w[par]).wait()
        pltpu.make_async_copy(self_bufs[par], self_out.at[pl.ds(0, K)],
                              sem_ws[par]).wait()


@jax.jit
def _sc_gather(nodes_w, neigh_w, features):
    mesh = plsc.VectorSubcoreMesh(core_axis_name="c", subcore_axis_name="s")
    f = pl.kernel(
        _sc_gather_body,
        out_type=(
            jax.ShapeDtypeStruct((B_PAD, D), jnp.float32),
            jax.ShapeDtypeStruct((B_PAD, D), jnp.float32),
        ),
        mesh=mesh,
        scratch_types=[
            pltpu.VMEM((NCHUNK, K), jnp.int32),
            pltpu.VMEM((NCHUNK, KS), jnp.int32),
            pltpu.VMEM((K, D), jnp.float32),
            pltpu.VMEM((K, D), jnp.float32),
            pltpu.VMEM((KS, D), jnp.float32),
            pltpu.VMEM((KS, D), jnp.float32),
            pltpu.VMEM((K, D), jnp.float32),
            pltpu.VMEM((K, D), jnp.float32),
        ] + [pltpu.SemaphoreType.DMA] * 8,
    )
    return f(nodes_w, neigh_w, features)


def _mm_body(ws_ref, wn_ref, self_ref, mean_ref, out_ref):
    a = lax.dot_general(ws_ref[...], self_ref[...],
                        (((1,), (1,)), ((), ())),
                        preferred_element_type=jnp.float32)
    b = lax.dot_general(wn_ref[...], mean_ref[...],
                        (((1,), (1,)), ((), ())),
                        preferred_element_type=jnp.float32)
    out_ref[...] = jnp.maximum(a + b, 0.0)


BT = 512


@jax.jit
def _tc_matmul(ws, wn, self_f, mean_f):
    grid = (B_PAD // BT,)
    return pl.pallas_call(
        _mm_body,
        grid=grid,
        in_specs=[
            pl.BlockSpec((E, D), lambda i: (0, 0)),
            pl.BlockSpec((E, D), lambda i: (0, 0)),
            pl.BlockSpec((BT, D), lambda i: (i, 0)),
            pl.BlockSpec((BT, D), lambda i: (i, 0)),
        ],
        out_specs=pl.BlockSpec((E, BT), lambda i: (0, i)),
        out_shape=jax.ShapeDtypeStruct((E, B), jnp.float32),
        compiler_params=pltpu.CompilerParams(
            dimension_semantics=("arbitrary",)),
    )(ws, wn, self_f, mean_f)


def kernel(nodes, neigh_idx, features, weight):
    nodes = nodes.astype(jnp.int32)
    neigh_idx = neigh_idx.astype(jnp.int32)
    nodes_w = jnp.pad(nodes, (0, B_PAD - B)).reshape(NW, NCHUNK, K)
    neigh_w = jnp.pad(neigh_idx, ((0, B_PAD - B), (0, 0))).reshape(
        NW, NCHUNK, KS)
    self_f, mean_f = _sc_gather(nodes_w, neigh_w, features)
    ws = weight[:, :D]
    wn = weight[:, D:]
    return _tc_matmul(ws, wn, self_f, mean_f)


# speedup vs baseline: 1.5875x; 1.0070x over previous
import jax
import jax.numpy as jnp
from jax import lax
from jax.experimental import pallas as pl
from jax.experimental.pallas import tpu as pltpu
from jax.experimental.pallas import tpu_sc as plsc

N_NODES = 50000
D = 512
E = 512
B = 10000
S = 10

NC, NS, L = 2, 16, 16
NW = NC * NS
B_PAD = 10240
RPW = B_PAD // NW
K = 8
NCHUNK = RPW // K
KS = K * S


def _sc_gather_body(nodes_hbm, neigh_hbm, feat_hbm,
                    self_out, mean_out,
                    sidx, nidx,
                    self_buf0, self_buf1, nbuf0, nbuf1, acc0, acc1,
                    sem_gs0, sem_gs1, sem_gn0, sem_gn1,
                    sem_w0, sem_w1, sem_ws0, sem_ws1):
    wid = lax.axis_index("s") * NC + lax.axis_index("c")
    base = wid * RPW
    self_bufs = (self_buf0, self_buf1)
    nbufs = (nbuf0, nbuf1)
    accs = (acc0, acc1)
    sem_gs = (sem_gs0, sem_gs1)
    sem_gn = (sem_gn0, sem_gn1)
    sem_w = (sem_w0, sem_w1)
    sem_ws = (sem_ws0, sem_ws1)

    pltpu.sync_copy(nodes_hbm.at[wid], sidx)
    pltpu.sync_copy(neigh_hbm.at[wid], nidx)

    inv_s = jnp.float32(1.0 / S)

    def issue_gathers(ch, par):
        pltpu.async_copy(feat_hbm.at[sidx.at[ch]], self_bufs[par], sem_gs[par])
        pltpu.async_copy(feat_hbm.at[nidx.at[ch]], nbufs[par], sem_gn[par])

    issue_gathers(0, 0)
    issue_gathers(1, 1)

    def pair_body(p, _):
        for par in range(2):
            ch = p * 2 + par
            row0 = base + ch * K
            sbuf, nbuf, acc = self_bufs[par], nbufs[par], accs[par]
            pltpu.make_async_copy(feat_hbm.at[sidx.at[ch]], sbuf,
                                  sem_gs[par]).wait()
            pltpu.make_async_copy(feat_hbm.at[nidx.at[ch]], nbuf,
                                  sem_gn[par]).wait()
            pltpu.async_copy(sbuf, self_out.at[pl.ds(row0, K)], sem_ws[par])
            @pl.when(ch >= 2)
            def _():
                pltpu.make_async_copy(
                    acc, mean_out.at[pl.ds(row0, K)], sem_w[par]).wait()

            def col_body(c, _):
                sl = pl.ds(c * L, L)
                for r in range(K):
                    a = nbuf[r * S, sl]
                    for j in range(1, S):
                        a = a + nbuf[r * S + j, sl]
                    acc[r, sl] = a * inv_s
                return 0
            lax.fori_loop(0, D // L, col_body, 0)

            pltpu.async_copy(acc, mean_out.at[pl.ds(row0, K)], sem_w[par])

            @pl.when(ch + 2 < NCHUNK)
            def _():
                pltpu.make_async_copy(sbuf, self_out.at[pl.ds(row0, K)],
                                      sem_ws[par]).wait()
                issue_gathers(ch + 2, par)
        return 0

    lax.fori_loop(0, NCHUNK // 2, pair_body, 0)

    for par in range(2):
        pltpu.make_async_copy(accs[par], mean_out.at[pl.ds(0, K)],
                              sem_w[par]).wait()
        pltpu.make_async_copy(self_bufs[par], self_out.at[pl.ds(0, K)],
                              sem_ws[par]).wait()


@jax.jit
def _sc_gather(nodes_w, neigh_w, features):
    mesh = plsc.VectorSubcoreMesh(core_axis_name="c", subcore_axis_name="s")
    f = pl.kernel(
        _sc_gather_body,
        out_type=(
            jax.ShapeDtypeStruct((B_PAD, D), jnp.float32),
            jax.ShapeDtypeStruct((B_PAD, D), jnp.float32),
        ),
        mesh=mesh,
        scratch_types=[
            pltpu.VMEM((NCHUNK, K), jnp.int32),
            pltpu.VMEM((NCHUNK, KS), jnp.int32),
            pltpu.VMEM((K, D), jnp.float32),
            pltpu.VMEM((K, D), jnp.float32),
            pltpu.VMEM((KS, D), jnp.float32),
            pltpu.VMEM((KS, D), jnp.float32),
            pltpu.VMEM((K, D), jnp.float32),
            pltpu.VMEM((K, D), jnp.float32),
        ] + [pltpu.SemaphoreType.DMA] * 8,
    )
    return f(nodes_w, neigh_w, features)


def _mm_body(ws_ref, wn_ref, self_ref, mean_ref, out_ref):
    a = lax.dot_general(ws_ref[...], self_ref[...],
                        (((1,), (1,)), ((), ())),
                        preferred_element_type=jnp.float32)
    b = lax.dot_general(wn_ref[...], mean_ref[...],
                        (((1,), (1,)), ((), ())),
                        preferred_element_type=jnp.float32)
    out_ref[...] = jnp.maximum(a + b, 0.0)


BT = 512


@jax.jit
def _tc_matmul(ws, wn, self_f, mean_f):
    grid = (B_PAD // BT,)
    return pl.pallas_call(
        _mm_body,
        grid=grid,
        in_specs=[
            pl.BlockSpec((E, D), lambda i: (0, 0)),
            pl.BlockSpec((E, D), lambda i: (0, 0)),
            pl.BlockSpec((BT, D), lambda i: (i, 0)),
            pl.BlockSpec((BT, D), lambda i: (i, 0)),
        ],
        out_specs=pl.BlockSpec((E, BT), lambda i: (0, i)),
        out_shape=jax.ShapeDtypeStruct((E, B), jnp.float32),
        compiler_params=pltpu.CompilerParams(
            dimension_semantics=("arbitrary",)),
    )(ws, wn, self_f, mean_f)


def kernel(nodes, neigh_idx, features, weight):
    nodes = nodes.astype(jnp.int32)
    neigh_idx = neigh_idx.astype(jnp.int32)
    nodes_w = jnp.pad(nodes, (0, B_PAD - B)).reshape(NW, NCHUNK, K)
    neigh_w = jnp.pad(neigh_idx, ((0, B_PAD - B), (0, 0))).reshape(
        NW, NCHUNK, KS)
    self_f, mean_f = _sc_gather(nodes_w, neigh_w, features)
    ws = weight[:, :D]
    wn = weight[:, D:]
    return _tc_matmul(ws, wn, self_f, mean_f)


# speedup vs baseline: 2.5530x; 1.6082x over previous
import jax
import jax.numpy as jnp
from jax import lax
from jax.experimental import pallas as pl
from jax.experimental.pallas import tpu as pltpu
from jax.experimental.pallas import tpu_sc as plsc

N_NODES = 50000
D = 512
E = 512
B = 10000
S = 10

NC, NS, L = 2, 16, 16
NW = NC * NS
B_PAD = 10240
RPW = B_PAD // NW
K = 4
NCHUNK = RPW // K
KS = K * S
NB = 4
PF = 3


def _sc_gather_body(nodes_hbm, neigh_hbm, feat_hbm,
                    self_out, mean_out,
                    sidx, nidx,
                    sbuf0, sbuf1, sbuf2, sbuf3,
                    nbuf0, nbuf1, nbuf2, nbuf3,
                    acc0, acc1, acc2, acc3,
                    gs0, gs1, gs2, gs3,
                    gn0, gn1, gn2, gn3,
                    w0, w1, w2, w3,
                    ws0, ws1, ws2, ws3):
    wid = lax.axis_index("s") * NC + lax.axis_index("c")
    base = wid * RPW
    sbufs = (sbuf0, sbuf1, sbuf2, sbuf3)
    nbufs = (nbuf0, nbuf1, nbuf2, nbuf3)
    accs = (acc0, acc1, acc2, acc3)
    sem_gs = (gs0, gs1, gs2, gs3)
    sem_gn = (gn0, gn1, gn2, gn3)
    sem_w = (w0, w1, w2, w3)
    sem_ws = (ws0, ws1, ws2, ws3)

    pltpu.sync_copy(nodes_hbm.at[wid], sidx)
    pltpu.sync_copy(neigh_hbm.at[wid], nidx)

    inv_s = jnp.float32(1.0 / S)

    def issue_gathers(ch, q):
        pltpu.async_copy(feat_hbm.at[sidx.at[ch]], sbufs[q], sem_gs[q])
        pltpu.async_copy(feat_hbm.at[nidx.at[ch]], nbufs[q], sem_gn[q])

    for ch in range(PF):
        issue_gathers(ch, ch)

    def group_body(g, _):
        for par in range(NB):
            ch = g * NB + par
            row0 = base + ch * K
            sbuf, nbuf, acc = sbufs[par], nbufs[par], accs[par]
            pltpu.make_async_copy(feat_hbm.at[sidx.at[ch]], sbuf,
                                  sem_gs[par]).wait()
            pltpu.make_async_copy(feat_hbm.at[nidx.at[ch]], nbuf,
                                  sem_gn[par]).wait()
            pltpu.async_copy(sbuf, self_out.at[pl.ds(row0, K)], sem_ws[par])
            @pl.when(ch >= NB)
            def _():
                pltpu.make_async_copy(
                    acc, mean_out.at[pl.ds(row0, K)], sem_w[par]).wait()

            def col_body(c, _):
                sl = pl.ds(c * L, L)
                for r in range(K):
                    a = nbuf[r * S, sl]
                    for j in range(1, S):
                        a = a + nbuf[r * S + j, sl]
                    acc[r, sl] = a * inv_s
                return 0
            lax.fori_loop(0, D // L, col_body, 0)

            pltpu.async_copy(acc, mean_out.at[pl.ds(row0, K)], sem_w[par])

            q = (par + PF) % NB
            @pl.when(ch + PF < NCHUNK)
            def _():
                @pl.when(ch + PF >= NB)
                def _():
                    pltpu.make_async_copy(
                        sbufs[q], self_out.at[pl.ds(row0, K)],
                        sem_ws[q]).wait()
                issue_gathers(ch + PF, q)
        return 0

    lax.fori_loop(0, NCHUNK // NB, group_body, 0)

    for par in range(NB):
        pltpu.make_async_copy(accs[par], mean_out.at[pl.ds(0, K)],
                              sem_w[par]).wait()
        pltpu.make_async_copy(sbufs[par], self_out.at[pl.ds(0, K)],
                              sem_ws[par]).wait()


@jax.jit
def _sc_gather(nodes_w, neigh_w, features):
    mesh = plsc.VectorSubcoreMesh(core_axis_name="c", subcore_axis_name="s")
    f = pl.kernel(
        _sc_gather_body,
        out_type=(
            jax.ShapeDtypeStruct((B_PAD, D), jnp.float32),
            jax.ShapeDtypeStruct((B_PAD, D), jnp.float32),
        ),
        mesh=mesh,
        scratch_types=[
            pltpu.VMEM((NCHUNK, K), jnp.int32),
            pltpu.VMEM((NCHUNK, KS), jnp.int32),
        ] + [pltpu.VMEM((K, D), jnp.float32)] * NB
          + [pltpu.VMEM((KS, D), jnp.float32)] * NB
          + [pltpu.VMEM((K, D), jnp.float32)] * NB
          + [pltpu.SemaphoreType.DMA] * (4 * NB),
    )
    return f(nodes_w, neigh_w, features)


def _mm_body(ws_ref, wn_ref, self_ref, mean_ref, out_ref):
    a = lax.dot_general(ws_ref[...], self_ref[...],
                        (((1,), (1,)), ((), ())),
                        preferred_element_type=jnp.float32)
    b = lax.dot_general(wn_ref[...], mean_ref[...],
                        (((1,), (1,)), ((), ())),
                        preferred_element_type=jnp.float32)
    out_ref[...] = jnp.maximum(a + b, 0.0)


BT = 512


@jax.jit
def _tc_matmul(ws, wn, self_f, mean_f):
    grid = (B_PAD // BT,)
    return pl.pallas_call(
        _mm_body,
        grid=grid,
        in_specs=[
            pl.BlockSpec((E, D), lambda i: (0, 0)),
            pl.BlockSpec((E, D), lambda i: (0, 0)),
            pl.BlockSpec((BT, D), lambda i: (i, 0)),
            pl.BlockSpec((BT, D), lambda i: (i, 0)),
        ],
        out_specs=pl.BlockSpec((E, BT), lambda i: (0, i)),
        out_shape=jax.ShapeDtypeStruct((E, B), jnp.float32),
        compiler_params=pltpu.CompilerParams(
            dimension_semantics=("arbitrary",)),
    )(ws, wn, self_f, mean_f)


def kernel(nodes, neigh_idx, features, weight):
    nodes = nodes.astype(jnp.int32)
    neigh_idx = neigh_idx.astype(jnp.int32)
    pad_n = B_PAD - B
    pad_rows = (jnp.arange(pad_n, dtype=jnp.int32) * 37) % N_NODES
    nodes_w = jnp.concatenate([nodes, pad_rows]).reshape(NW, NCHUNK, K)
    pad_rows2 = (jnp.arange(pad_n * S, dtype=jnp.int32) * 37) % N_NODES
    neigh_w = jnp.concatenate([neigh_idx.reshape(-1), pad_rows2]).reshape(
        NW, NCHUNK, KS)
    self_f, mean_f = _sc_gather(nodes_w, neigh_w, features)
    ws = weight[:, :D]
    wn = weight[:, D:]
    return _tc_matmul(ws, wn, self_f, mean_f)
